# Initial kernel scaffold; baseline (speedup 1.0000x reference)
import jax
import jax.numpy as jnp
from jax.experimental import pallas as pl


def kernel(sents, lengths, forgetting_factor):
    raise NotImplementedError("write your pallas kernel here")


# speedup vs baseline: 16.1841x; 16.1841x over previous
import functools

import jax
import jax.numpy as jnp
from jax import lax
from jax.experimental import pallas as pl
from jax.experimental.pallas import tpu as pltpu
from jax.experimental.pallas import tpu_sc as plsc

_VOCAB = 128
_B, _S, _W = 64, 256, 16
_R = _B * _S
_NC, _NS = 2, 16
_NW = _NC * _NS
_RPW = _R // _NW


def _fofe_body(ids_hbm, pow_hbm, out_hbm, ids_v, pow_v, out_v):
    wid = lax.axis_index("s") * _NC + lax.axis_index("c")
    base = wid * _RPW

    pltpu.sync_copy(ids_hbm.at[pl.ds(base, _RPW)], ids_v)
    pltpu.sync_copy(pow_hbm, pow_v)
    pvec = pow_v[...]
    zero16 = jnp.zeros((16,), jnp.float32)

    def row_body(r, carry):
        for c in range(_VOCAB // 16):
            out_v[r, pl.ds(c * 16, 16)] = zero16
        idx = ids_v[r, :]
        rowv = jnp.full((16,), r, jnp.int32)
        plsc.addupdate_scatter(out_v, [rowv, idx], pvec)
        return carry

    lax.fori_loop(0, _RPW, row_body, 0)
    pltpu.sync_copy(out_v, out_hbm.at[pl.ds(base, _RPW)])


@jax.jit
def _fofe(ids, powers):
    mesh = plsc.VectorSubcoreMesh(core_axis_name="c", subcore_axis_name="s")
    run = functools.partial(
        pl.kernel,
        mesh=mesh,
        out_type=jax.ShapeDtypeStruct((_R, _VOCAB), jnp.float32),
        scratch_types=[
            pltpu.VMEM((_RPW, _W), jnp.int32),
            pltpu.VMEM((_W,), jnp.float32),
            pltpu.VMEM((_RPW, _VOCAB), jnp.float32),
        ],
        compiler_params=pltpu.CompilerParams(
            needs_layout_passes=False, use_tc_tiling_on_sc=False
        ),
    )(_fofe_body)
    return run(ids, powers)


def kernel(sents, lengths, forgetting_factor):
    f = forgetting_factor[0]
    powers = f ** jnp.arange(_W - 1, -1, -1, dtype=jnp.float32)
    out = _fofe(sents.reshape(_R, _W), powers)
    return (out.reshape(_B, _S, _VOCAB), lengths)


# speedup vs baseline: 16.7385x; 1.0343x over previous
import functools

import jax
import jax.numpy as jnp
from jax import lax
from jax.experimental import pallas as pl
from jax.experimental.pallas import tpu as pltpu
from jax.experimental.pallas import tpu_sc as plsc

_VOCAB = 128
_B, _S, _W = 64, 256, 16
_R = _B * _S
_NC, _NS = 2, 16
_NW = _NC * _NS
_RPW = _R // _NW


_NB = 4
_BR = _RPW // _NB


def _fofe_body(ids_hbm, pow_hbm, out_hbm, ids_v, pow_v, out_v, sem):
    wid = lax.axis_index("s") * _NC + lax.axis_index("c")
    base = wid * _RPW

    pltpu.sync_copy(ids_hbm.at[pl.ds(base, _RPW)], ids_v)
    pltpu.sync_copy(pow_hbm, pow_v)
    pvec = pow_v[...]
    zero16 = jnp.zeros((16,), jnp.float32)

    def row_body(r, carry):
        for c in range(_VOCAB // 16):
            out_v[r, pl.ds(c * 16, 16)] = zero16
        idx = ids_v[r, :]
        rowv = jnp.full((16,), r, jnp.int32)
        plsc.addupdate_scatter(out_v, [rowv, idx], pvec)
        return carry

    copies = []
    for blk in range(_NB):
        lax.fori_loop(blk * _BR, (blk + 1) * _BR, row_body, 0, unroll=8)
        cp = pltpu.make_async_copy(
            out_v.at[pl.ds(blk * _BR, _BR)],
            out_hbm.at[pl.ds(base + blk * _BR, _BR)],
            sem,
        )
        cp.start()
        copies.append(cp)
    for cp in copies:
        cp.wait()


@jax.jit
def _fofe(ids, powers):
    mesh = plsc.VectorSubcoreMesh(core_axis_name="c", subcore_axis_name="s")
    run = functools.partial(
        pl.kernel,
        mesh=mesh,
        out_type=jax.ShapeDtypeStruct((_R, _VOCAB), jnp.float32),
        scratch_types=[
            pltpu.VMEM((_RPW, _W), jnp.int32),
            pltpu.VMEM((_W,), jnp.float32),
            pltpu.VMEM((_RPW, _VOCAB), jnp.float32),
            pltpu.SemaphoreType.DMA,
        ],
        compiler_params=pltpu.CompilerParams(
            needs_layout_passes=False, use_tc_tiling_on_sc=False
        ),
    )(_fofe_body)
    return run(ids, powers)


def kernel(sents, lengths, forgetting_factor):
    f = forgetting_factor[0]
    powers = f ** jnp.arange(_W - 1, -1, -1, dtype=jnp.float32)
    out = _fofe(sents.reshape(_R, _W), powers)
    return (out.reshape(_B, _S, _VOCAB), lengths)
